# Initial kernel scaffold; baseline (speedup 1.0000x reference)
import jax
import jax.numpy as jnp
from jax.experimental import pallas as pl


def kernel(feat, edge_index, params):
    raise NotImplementedError("write your pallas kernel here")


# speedup vs baseline: 2.3132x; 2.3132x over previous
import functools

import jax
import jax.numpy as jnp
from jax import lax
from jax.experimental import pallas as pl
from jax.experimental.pallas import tpu as pltpu
from jax.experimental.pallas import tpu_sc as plsc

N1 = 10000
E = 320000
HID = 128
RATIO = 0.8
K1 = int(RATIO * N1)
K2 = int(RATIO * K1)
NTRASH = 32
NPAD = 10240
NC, NS, NW = 2, 16, 32
ROWS_PT = NPAD // NS
BR = 1024
NBLK = NPAD // BR


@functools.lru_cache(maxsize=None)
def _make_prop(D, NE, K):
    ne_w = NE // NW
    nch = ne_w // K
    assert ne_w % K == 0 and K % 8 == 0 and K <= 128
    mesh = plsc.VectorSubcoreMesh(core_axis_name="c", subcore_axis_name="s")

    def body(u_hbm, gi_hbm, si_hbm, zrow_hbm, out_hbm, gi_v, si_v, rows_v,
             acc_sh, sem):
        c = lax.axis_index("c")
        s = lax.axis_index("s")
        wid = s * NC + c
        pltpu.sync_copy(zrow_hbm, acc_sh.at[pl.ds(s * ROWS_PT, ROWS_PT)])
        plsc.subcore_barrier()
        base0 = wid * ne_w

        def chunk(i, carry):
            base = base0 + i * K
            pltpu.sync_copy(gi_hbm.at[pl.ds(base, K)], gi_v)
            pltpu.sync_copy(si_hbm.at[pl.ds(base, K)], si_v)
            pltpu.async_copy(u_hbm.at[gi_v], rows_v, sem).wait()
            pltpu.sync_copy(rows_v, acc_sh.at[si_v], add=True)
            return carry

        lax.fori_loop(0, nch, chunk, 0)
        plsc.subcore_barrier()
        pltpu.sync_copy(acc_sh.at[pl.ds(s * ROWS_PT, ROWS_PT)],
                        out_hbm.at[pl.ds(c * NPAD + s * ROWS_PT, ROWS_PT)])

    return pl.kernel(
        body,
        out_type=jax.ShapeDtypeStruct((NC * NPAD, D), jnp.float32),
        mesh=mesh,
        scratch_types=[
            pltpu.VMEM((K,), jnp.int32),
            pltpu.VMEM((K,), jnp.int32),
            pltpu.VMEM((K, D), jnp.float32),
            pltpu.VMEM_SHARED((NPAD, D), jnp.float32),
            pltpu.SemaphoreType.DMA,
        ],
        compiler_params=pltpu.CompilerParams(
            use_tc_tiling_on_sc=(D % 128 == 0)),
    )


def _prop(u, gi, si, D, NE=E, K=80):
    z = jnp.zeros((ROWS_PT, D), jnp.float32)
    out = _make_prop(D, NE, K)(u, gi, si, z)
    return out.reshape(NC, NPAD, D)


def _dot(a, b):
    return lax.dot_general(a, b, (((1,), (0,)), ((), ())),
                           preferred_element_type=jnp.float32)


_VSPEC = pl.BlockSpec((BR, 1), lambda i: (i, 0))
_XSPEC = pl.BlockSpec((BR, HID), lambda i: (i, 0))
_YSPEC = pl.BlockSpec((NC, BR, HID), lambda i: (0, i, 0))


@functools.lru_cache(maxsize=None)
def _make_stage(act, with_acc):
    def body(*refs):
        if with_acc:
            (x0, y1, y2, s, m, w, b, a, acc, o) = refs
        else:
            (x0, y1, y2, s, m, w, b, a, o) = refs
            acc = None
        sv = s[...]
        h = _dot(x0[...], w[0])
        h = h + _dot((y1[0] + y1[1]) * sv, w[1])
        h = h + _dot((y2[0] + y2[1]) * sv, w[2])
        if acc is not None:
            h = h + acc[...]
        h = (h + b[...]) * m[...]
        if act == 1:
            h = jnp.maximum(h, 0.0)
        elif act == 2:
            h = jnp.where(h >= 0.0, h, a[0, 0] * h)
        o[...] = h

    in_specs = [
        _XSPEC, _YSPEC, _YSPEC, _VSPEC, _VSPEC,
        pl.BlockSpec((3, HID, HID), lambda i: (0, 0, 0)),
        pl.BlockSpec((1, HID), lambda i: (0, 0)),
        pl.BlockSpec((1, 1), lambda i: (0, 0)),
    ]
    if with_acc:
        in_specs.append(_XSPEC)
    return pl.pallas_call(
        body,
        grid=(NBLK,),
        in_specs=in_specs,
        out_specs=_XSPEC,
        out_shape=jax.ShapeDtypeStruct((NPAD, HID), jnp.float32),
    )


def _tc_stage(x0, y1p, y2p, svec, mvec, Ws3, bias, act=0, a=0.0, acc=None):
    aarr = jnp.asarray(a, jnp.float32).reshape(1, 1)
    barr = bias.reshape(1, HID)
    args = [x0, y1p, y2p, svec, mvec, Ws3, barr, aarr]
    if acc is not None:
        args.append(acc)
    return _make_stage(act, acc is not None)(*args)


@functools.lru_cache(maxsize=None)
def _make_scale(dims_key):
    def body(*refs):
        xs, s, o = refs[:-2], refs[-2], refs[-1]
        accv = None
        for r, nd in zip(xs, dims_key):
            v = r[...]
            if nd == 3:
                v = v[0] + v[1]
            accv = v if accv is None else accv + v
        o[...] = accv * s[...]

    in_specs = [(_YSPEC if nd == 3 else _XSPEC) for nd in dims_key] + [_VSPEC]
    return pl.pallas_call(
        body,
        grid=(NBLK,),
        in_specs=in_specs,
        out_specs=_XSPEC,
        out_shape=jax.ShapeDtypeStruct((NPAD, HID), jnp.float32),
    )


def _tc_scale(xs, svec):
    dims_key = tuple(x.ndim for x in xs)
    return _make_scale(dims_key)(*xs, svec)


@functools.lru_cache(maxsize=None)
def _make_affine():
    def body(x, w, b, m, o):
        o[...] = (_dot(x[...], w[...]) + b[...]) * m[...]

    return pl.pallas_call(
        body,
        grid=(NBLK,),
        in_specs=[
            _XSPEC,
            pl.BlockSpec((HID, HID), lambda i: (0, 0)),
            pl.BlockSpec((1, HID), lambda i: (0, 0)),
            _VSPEC,
        ],
        out_specs=_XSPEC,
        out_shape=jax.ShapeDtypeStruct((NPAD, HID), jnp.float32),
    )


def _tc_affine(x, w, bias, mvec):
    return _make_affine()(x, w, bias.reshape(1, HID), mvec)


@functools.lru_cache(maxsize=None)
def _make_summary(kdiv):
    def body(h, wd, o):
        ssum = jnp.sum(h[...], axis=0, keepdims=True) * (1.0 / kdiv)
        sig = jax.nn.sigmoid(ssum)
        o[...] = lax.dot_general(sig, wd[...], (((1,), (1,)), ((), ())),
                                 preferred_element_type=jnp.float32)

    return pl.pallas_call(
        body,
        in_specs=[
            pl.BlockSpec((NPAD, HID), lambda: (0, 0)),
            pl.BlockSpec((HID, HID), lambda: (0, 0)),
        ],
        out_specs=pl.BlockSpec((1, HID), lambda: (0, 0)),
        out_shape=jax.ShapeDtypeStruct((1, HID), jnp.float32),
    )


@functools.lru_cache(maxsize=None)
def _make_matvec():
    def body(h, v, o):
        o[...] = _dot(h[...], v[...])

    return pl.pallas_call(
        body,
        grid=(NBLK,),
        in_specs=[_XSPEC, pl.BlockSpec((HID, 1), lambda i: (0, 0))],
        out_specs=_VSPEC,
        out_shape=jax.ShapeDtypeStruct((NPAD, 1), jnp.float32),
    )


def _matvec(h, v128):
    return _make_matvec()(h, v128.reshape(HID, 1))[:, 0]


def _stack_ws(Ws):
    ws = list(Ws)
    if len(ws) == 2:
        ws.append(jnp.zeros_like(ws[1]))
    return jnp.stack(ws)


def _tag_run(x0, gi, si, sA, sAB, sB, p, mvec, act=0, a=0.0):
    u0 = _tc_scale([x0], sA)
    y1 = _prop(u0, gi, si, HID)
    if len(p['Ws']) == 3:
        u1 = _tc_scale([y1], sAB)
        y2 = _prop(u1, gi, si, HID)
    else:
        y2 = jnp.zeros_like(y1)
    return _tc_stage(x0, y1, y2, sB, mvec, _stack_ws(p['Ws']), p['b'], act, a)


def _degs(gi, si):
    ones16 = jnp.ones((NPAD, 16), jnp.float32)
    out = _prop(ones16, gi, si, 16)
    return out[0, :, 0] + out[1, :, 0]


def _perm_copy(x, pext, onescol):
    yp = _prop(x, pext, jnp.arange(NPAD, dtype=jnp.int32), HID, NE=NPAD, K=64)
    return _tc_scale([yp], onescol)


def _col(v):
    return v.reshape(NPAD, 1)


def _tag_xla(x, src, dst, w, n, Ws, b):
    deg_i = jnp.maximum(jnp.zeros(n, x.dtype).at[dst].add(w), 1.0)
    deg_o = jnp.maximum(jnp.zeros(n, x.dtype).at[src].add(w), 1.0)
    norm = w / jnp.sqrt(jnp.take(deg_o, src) * jnp.take(deg_i, dst))
    h = x @ Ws[0]
    xk = x
    for Wk in Ws[1:]:
        xk = jnp.zeros((n, xk.shape[1]), x.dtype).at[dst].add(
            norm[:, None] * jnp.take(xk, src, axis=0))
        h = h + xk @ Wk
    return h + b


def _prelu_xla(x, a):
    return jnp.where(x >= 0, x, a * x)


def kernel(feat, edge_index, params):
    src = edge_index[0]
    dst = edge_index[1]
    featP = jnp.pad(feat, ((0, NPAD - N1), (0, 0)))
    ones_v = (jnp.arange(NPAD) < N1).astype(jnp.float32)
    onescol = _col(ones_v)

    deg_i1 = _degs(src, dst)
    deg_o1 = _degs(dst, src)
    sA1 = _col(ones_v * lax.rsqrt(jnp.maximum(deg_o1, 1.0)))
    sB1 = _col(lax.rsqrt(jnp.maximum(deg_i1, 1.0)))
    sAB1 = _col(sA1[:, 0] * sB1[:, 0] * ones_v)

    w1 = jnp.ones((E,), jnp.float32)
    p = params['is1']
    f1x = _tag_xla(feat, src, dst, w1, N1, params['sg1']['Ws'],
                   params['sg1']['b'])
    perm1 = jax.random.permutation(jax.random.key(1), N1)
    f1nx = jnp.take(f1x, perm1, axis=0)
    h1x = _prelu_xla(_tag_xla(f1x, src, dst, w1, N1, p['Ws'], p['b']), p['a'])
    hn1x = _prelu_xla(_tag_xla(f1nx, src, dst, w1, N1, p['Ws'], p['b']),
                      p['a'])
    summary1 = jax.nn.sigmoid(jnp.mean(h1x, axis=0))
    sc1p = (h1x @ p['Wd']) @ summary1
    sc1n = (hn1x @ p['Wd']) @ summary1
    logit1 = jnp.concatenate([sc1p, sc1n])
    scores1 = jax.nn.sigmoid(sc1p)
    sel1 = jnp.argsort(-scores1)[:K1]
    f1 = jnp.pad(f1x, ((0, NPAD - N1), (0, 0)))
    feat_origin = f1
    down1 = jnp.pad(h1x, ((0, NPAD - N1), (0, 0)))

    maskA_b = jnp.zeros((N1,), bool).at[sel1].set(True)
    maskA = jnp.pad(maskA_b.astype(jnp.float32), (0, NPAD - N1))
    mAcol = _col(maskA)
    rank1 = jnp.zeros((N1,), jnp.int32).at[sel1].set(
        jnp.arange(K1, dtype=jnp.int32))
    ar = jnp.arange(E, dtype=jnp.int32)
    trashv = N1 + (ar & (NTRASH - 1))
    vA = maskA_b[src] & maskA_b[dst]
    srcA = jnp.where(vA, src, trashv)
    dstA = jnp.where(vA, dst, trashv)
    deg_iA = _degs(src, dstA)
    deg_oA = _degs(dst, srcA)
    sA2 = _col(maskA * lax.rsqrt(jnp.maximum(deg_oA, 1.0)))
    sB2 = _col(lax.rsqrt(jnp.maximum(deg_iA, 1.0)))
    sAB2 = _col(sA2[:, 0] * sB2[:, 0])

    f2p = _tc_scale([f1], _col(jnp.pad(scores1, (0, NPAD - N1)) * maskA))
    f2 = _tag_run(f2p, src, dstA, sA2, sAB2, sB2, params['sg2'], mAcol)

    p = params['is2']
    h2 = _tag_run(f2, src, dstA, sA2, sAB2, sB2, p, mAcol, act=2, a=p['a'])
    perm2 = jax.random.permutation(jax.random.key(2), K1)
    p2core = sel1[perm2[rank1]].astype(jnp.int32)
    p2 = jnp.where(maskA_b, p2core, jnp.arange(N1, dtype=jnp.int32))
    p2ext = jnp.concatenate([p2, jnp.arange(N1, NPAD, dtype=jnp.int32)])
    f2n = _perm_copy(f2, p2ext, onescol)
    hn2 = _tag_run(f2n, src, dstA, sA2, sAB2, sB2, p, mAcol, act=2, a=p['a'])
    v2 = _make_summary(K1)(h2, p['Wd'])
    sc2p = _matvec(h2, v2)
    sc2n = _matvec(hn2, v2)
    logit2 = jnp.concatenate([sc2p[sel1], sc2n[sel1]])
    scores2 = jax.nn.sigmoid(sc2p[sel1])
    sel2 = jnp.argsort(-scores2)[:K2]
    down2 = h2

    sel2n1 = sel1[sel2]
    maskB_b = jnp.zeros((N1,), bool).at[sel2n1].set(True)
    maskB = jnp.pad(maskB_b.astype(jnp.float32), (0, NPAD - N1))
    mBcol = _col(maskB)
    vB = maskB_b[src] & maskB_b[dst]
    srcB = jnp.where(vB, src, trashv)
    dstB = jnp.where(vB, dst, trashv)
    deg_iB = _degs(src, dstB)
    deg_oB = _degs(dst, srcB)
    sA3 = _col(maskB * lax.rsqrt(jnp.maximum(deg_oB, 1.0)))
    sB3 = _col(lax.rsqrt(jnp.maximum(deg_iB, 1.0)))
    sAB3 = _col(sA3[:, 0] * sB3[:, 0])

    sc2emb = jax.nn.sigmoid(sc2p) * maskB
    f3p = _tc_scale([f2], _col(sc2emb))

    f1a = _tag_run(f1, src, dst, sA1, sAB1, sB1, params['s1l1'], onescol, act=1)
    f2a = _tag_run(f2, src, dstA, sA2, sAB2, sB2, params['s2l1'], mAcol, act=1)
    f3a = _tag_run(f3p, src, dstB, sA3, sAB3, sB3, params['s3l1'], mBcol, act=1)

    WT = params['cf1']['W'].T
    bcf = params['cf1']['b']
    f1b = _tc_affine(f1a, WT, bcf, onescol)
    f2b = _tc_affine(f2a, WT, bcf, mAcol)
    f3b = _tc_affine(f3a, WT, bcf, mBcol)

    f1c = _tag_run(f1b, src, dst, sA1, sAB1, sB1, params['s1l2'], onescol, act=1)
    f2c = _tag_run(f2b, src, dstA, sA2, sAB2, sB2, params['s2l2'], mAcol, act=1)
    f3c = _tag_run(f3b, src, dstB, sA3, sAB3, sB3, params['s3l2'], mBcol, act=1)

    f3o = _tag_run(f3c, src, dstA, sA2, sAB2, sB2, params['end_unpool_s32'],
                   mAcol)
    S = _tc_scale([f2c, f3o, down2], onescol)
    f2out = _tag_run(S, src, dst, sA1, sAB1, sB1, params['end_unpool_s21'],
                     onescol)

    L = _tc_scale([f1c, f2out, down1], onescol)
    R = feat_origin
    WsL = [w[:HID] for w in params['end_gcn']['Ws']]
    WsR = [w[HID:] for w in params['end_gcn']['Ws']]
    uL = _tc_scale([L], sA1)
    uR = _tc_scale([R], sA1)
    y1L = _prop(uL, src, dst, HID)
    y1R = _prop(uR, src, dst, HID)
    u2L = _tc_scale([y1L], sAB1)
    u2R = _tc_scale([y1R], sAB1)
    y2L = _prop(u2L, src, dst, HID)
    y2R = _prop(u2R, src, dst, HID)
    zb = jnp.zeros((HID,), jnp.float32)
    hhalf = _tc_stage(L, y1L, y2L, sB1, onescol, jnp.stack(WsL), zb)
    fagg = _tc_stage(R, y1R, y2R, sB1, onescol, jnp.stack(WsR),
                     params['end_gcn']['b'], acc=hhalf)

    return fagg[:N1], logit1, logit2


# speedup vs baseline: 2.8867x; 1.2479x over previous
import functools

import jax
import jax.numpy as jnp
from jax import lax
from jax.experimental import pallas as pl
from jax.experimental.pallas import tpu as pltpu
from jax.experimental.pallas import tpu_sc as plsc

N1 = 10000
E = 320000
HID = 128
RATIO = 0.8
K1 = int(RATIO * N1)
K2 = int(RATIO * K1)
NTRASH = 32
NPAD = 10240
NC, NS, NW = 2, 16, 32
ROWS_PT = NPAD // NS
BR = 1024
NBLK = NPAD // BR


@functools.lru_cache(maxsize=None)
def _make_prop(D, NE, K):
    ne_w = NE // NW
    nch = ne_w // K
    assert ne_w % K == 0 and K % 8 == 0 and K <= 128
    mesh = plsc.VectorSubcoreMesh(core_axis_name="c", subcore_axis_name="s")

    def body(u_hbm, gi_hbm, si_hbm, zrow_hbm, out_hbm, gi_all, si_v, rows_v,
             acc_sh, sem_i, sem_g, sem_s):
        c = lax.axis_index("c")
        s = lax.axis_index("s")
        wid = s * NC + c
        base0 = wid * ne_w
        pltpu.sync_copy(zrow_hbm, acc_sh.at[pl.ds(s * ROWS_PT, ROWS_PT)])
        pltpu.sync_copy(gi_hbm.at[pl.ds(base0, ne_w)], gi_all)
        pltpu.sync_copy(si_hbm.at[pl.ds(base0, K)], si_v.at[0])
        plsc.subcore_barrier()
        pltpu.async_copy(u_hbm.at[gi_all.at[pl.ds(0, K)]], rows_v.at[0],
                         sem_g).wait()

        def chunk(i, carry):
            b = lax.rem(i, 2)

            @pl.when(i >= 1)
            def _():
                pltpu.make_async_copy(si_hbm.at[pl.ds(base0, K)],
                                      si_v.at[b], sem_i).wait()

            pltpu.async_copy(rows_v.at[b], acc_sh.at[si_v.at[b]], sem_s,
                             add=True)

            @pl.when(i + 1 < nch)
            def _():
                @pl.when(i >= 1)
                def _():
                    pltpu.make_async_copy(rows_v.at[1 - b],
                                          acc_sh.at[si_v.at[1 - b]],
                                          sem_s).wait()
                pltpu.async_copy(si_hbm.at[pl.ds(base0 + (i + 1) * K, K)],
                                 si_v.at[1 - b], sem_i)
                pltpu.async_copy(u_hbm.at[gi_all.at[pl.ds((i + 1) * K, K)]],
                                 rows_v.at[1 - b], sem_g).wait()

            return carry

        lax.fori_loop(0, nch, chunk, 0)
        if nch > 1:
            pltpu.make_async_copy(rows_v.at[0], acc_sh.at[si_v.at[0]],
                                  sem_s).wait()
        pltpu.make_async_copy(rows_v.at[0], acc_sh.at[si_v.at[0]],
                              sem_s).wait()
        plsc.subcore_barrier()
        pltpu.sync_copy(acc_sh.at[pl.ds(s * ROWS_PT, ROWS_PT)],
                        out_hbm.at[pl.ds(c * NPAD + s * ROWS_PT, ROWS_PT)])

    return pl.kernel(
        body,
        out_type=jax.ShapeDtypeStruct((NC * NPAD, D), jnp.float32),
        mesh=mesh,
        scratch_types=[
            pltpu.VMEM((ne_w,), jnp.int32),
            pltpu.VMEM((2, K), jnp.int32),
            pltpu.VMEM((2, K, D), jnp.float32),
            pltpu.VMEM_SHARED((NPAD, D), jnp.float32),
            pltpu.SemaphoreType.DMA,
            pltpu.SemaphoreType.DMA,
            pltpu.SemaphoreType.DMA,
        ],
        compiler_params=pltpu.CompilerParams(
            use_tc_tiling_on_sc=(D % 128 == 0)),
    )


def _prop(u, gi, si, D):
    K = 128
    ne0 = gi.shape[0]
    ne = -(-ne0 // (NW * K)) * (NW * K)
    if ne != ne0:
        ar = jnp.arange(ne - ne0, dtype=jnp.int32)
        gi = jnp.concatenate([gi, ar % N1])
        si = jnp.concatenate([si, N1 + (ar % NTRASH)])
    nch = ne // NW // K
    z = jnp.zeros((ROWS_PT, D), jnp.float32)
    out = _make_prop(D, ne, K)(u, gi, si, z)
    return out.reshape(NC, NPAD, D)


def _dot(a, b):
    return lax.dot_general(a, b, (((1,), (0,)), ((), ())),
                           preferred_element_type=jnp.float32)


_VSPEC = pl.BlockSpec((BR, 1), lambda i: (i, 0))
_XSPEC = pl.BlockSpec((BR, HID), lambda i: (i, 0))
_YSPEC = pl.BlockSpec((NC, BR, HID), lambda i: (0, i, 0))


@functools.lru_cache(maxsize=None)
def _make_stage(act, with_acc):
    def body(*refs):
        if with_acc:
            (x0, y1, y2, s, m, w, b, a, acc, o) = refs
        else:
            (x0, y1, y2, s, m, w, b, a, o) = refs
            acc = None
        sv = s[...]
        h = _dot(x0[...], w[0])
        h = h + _dot((y1[0] + y1[1]) * sv, w[1])
        h = h + _dot((y2[0] + y2[1]) * sv, w[2])
        if acc is not None:
            h = h + acc[...]
        h = (h + b[...]) * m[...]
        if act == 1:
            h = jnp.maximum(h, 0.0)
        elif act == 2:
            h = jnp.where(h >= 0.0, h, a[0, 0] * h)
        o[...] = h

    in_specs = [
        _XSPEC, _YSPEC, _YSPEC, _VSPEC, _VSPEC,
        pl.BlockSpec((3, HID, HID), lambda i: (0, 0, 0)),
        pl.BlockSpec((1, HID), lambda i: (0, 0)),
        pl.BlockSpec((1, 1), lambda i: (0, 0)),
    ]
    if with_acc:
        in_specs.append(_XSPEC)
    return pl.pallas_call(
        body,
        grid=(NBLK,),
        in_specs=in_specs,
        out_specs=_XSPEC,
        out_shape=jax.ShapeDtypeStruct((NPAD, HID), jnp.float32),
    )


def _tc_stage(x0, y1p, y2p, svec, mvec, Ws3, bias, act=0, a=0.0, acc=None):
    aarr = jnp.asarray(a, jnp.float32).reshape(1, 1)
    barr = bias.reshape(1, HID)
    args = [x0, y1p, y2p, svec, mvec, Ws3, barr, aarr]
    if acc is not None:
        args.append(acc)
    return _make_stage(act, acc is not None)(*args)


@functools.lru_cache(maxsize=None)
def _make_scale(dims_key):
    def body(*refs):
        xs, s, o = refs[:-2], refs[-2], refs[-1]
        accv = None
        for r, nd in zip(xs, dims_key):
            v = r[...]
            if nd == 3:
                v = v[0] + v[1]
            accv = v if accv is None else accv + v
        o[...] = accv * s[...]

    in_specs = [(_YSPEC if nd == 3 else _XSPEC) for nd in dims_key] + [_VSPEC]
    return pl.pallas_call(
        body,
        grid=(NBLK,),
        in_specs=in_specs,
        out_specs=_XSPEC,
        out_shape=jax.ShapeDtypeStruct((NPAD, HID), jnp.float32),
    )


def _tc_scale(xs, svec):
    dims_key = tuple(x.ndim for x in xs)
    return _make_scale(dims_key)(*xs, svec)


@functools.lru_cache(maxsize=None)
def _make_affine():
    def body(x, w, b, m, o):
        o[...] = (_dot(x[...], w[...]) + b[...]) * m[...]

    return pl.pallas_call(
        body,
        grid=(NBLK,),
        in_specs=[
            _XSPEC,
            pl.BlockSpec((HID, HID), lambda i: (0, 0)),
            pl.BlockSpec((1, HID), lambda i: (0, 0)),
            _VSPEC,
        ],
        out_specs=_XSPEC,
        out_shape=jax.ShapeDtypeStruct((NPAD, HID), jnp.float32),
    )


def _tc_affine(x, w, bias, mvec):
    return _make_affine()(x, w, bias.reshape(1, HID), mvec)


@functools.lru_cache(maxsize=None)
def _make_summary(kdiv):
    def body(h, wd, o):
        ssum = jnp.sum(h[...], axis=0, keepdims=True) * (1.0 / kdiv)
        sig = jax.nn.sigmoid(ssum)
        o[...] = lax.dot_general(sig, wd[...], (((1,), (1,)), ((), ())),
                                 preferred_element_type=jnp.float32)

    return pl.pallas_call(
        body,
        in_specs=[
            pl.BlockSpec((NPAD, HID), lambda: (0, 0)),
            pl.BlockSpec((HID, HID), lambda: (0, 0)),
        ],
        out_specs=pl.BlockSpec((1, HID), lambda: (0, 0)),
        out_shape=jax.ShapeDtypeStruct((1, HID), jnp.float32),
    )


@functools.lru_cache(maxsize=None)
def _make_matvec():
    def body(h, v, o):
        o[...] = _dot(h[...], v[...])

    return pl.pallas_call(
        body,
        grid=(NBLK,),
        in_specs=[_XSPEC, pl.BlockSpec((HID, 1), lambda i: (0, 0))],
        out_specs=_VSPEC,
        out_shape=jax.ShapeDtypeStruct((NPAD, 1), jnp.float32),
    )


def _matvec(h, v128):
    return _make_matvec()(h, v128.reshape(HID, 1))[:, 0]


def _stack_ws(Ws):
    ws = list(Ws)
    if len(ws) == 2:
        ws.append(jnp.zeros_like(ws[1]))
    return jnp.stack(ws)


def _tag_run(x0, gi, si, sA, sAB, sB, p, mvec, act=0, a=0.0):
    u0 = _tc_scale([x0], sA)
    y1 = _prop(u0, gi, si, HID)
    if len(p['Ws']) == 3:
        u1 = _tc_scale([y1], sAB)
        y2 = _prop(u1, gi, si, HID)
    else:
        y2 = jnp.zeros_like(y1)
    return _tc_stage(x0, y1, y2, sB, mvec, _stack_ws(p['Ws']), p['b'], act, a)


def _degs(gi, si):
    ones16 = jnp.ones((NPAD, 16), jnp.float32)
    out = _prop(ones16, gi, si, 16)
    return out[0, :, 0] + out[1, :, 0]


def _perm_copy(x, pext, onescol):
    yp = _prop(x, pext, jnp.arange(NPAD, dtype=jnp.int32), HID)
    return _tc_scale([yp], onescol)


def _col(v):
    return v.reshape(NPAD, 1)


def _tag_xla(x, src, dst, w, n, Ws, b):
    deg_i = jnp.maximum(jnp.zeros(n, x.dtype).at[dst].add(w), 1.0)
    deg_o = jnp.maximum(jnp.zeros(n, x.dtype).at[src].add(w), 1.0)
    norm = w / jnp.sqrt(jnp.take(deg_o, src) * jnp.take(deg_i, dst))
    h = x @ Ws[0]
    xk = x
    for Wk in Ws[1:]:
        xk = jnp.zeros((n, xk.shape[1]), x.dtype).at[dst].add(
            norm[:, None] * jnp.take(xk, src, axis=0))
        h = h + xk @ Wk
    return h + b


def _prelu_xla(x, a):
    return jnp.where(x >= 0, x, a * x)


def kernel(feat, edge_index, params):
    src = edge_index[0]
    dst = edge_index[1]
    featP = jnp.pad(feat, ((0, NPAD - N1), (0, 0)))
    ones_v = (jnp.arange(NPAD) < N1).astype(jnp.float32)
    onescol = _col(ones_v)

    w1 = jnp.ones((E,), jnp.float32)
    deg_i1 = jnp.pad(jnp.zeros((N1,), jnp.float32).at[dst].add(w1),
                     (0, NPAD - N1))
    deg_o1 = jnp.pad(jnp.zeros((N1,), jnp.float32).at[src].add(w1),
                     (0, NPAD - N1))
    sA1 = _col(ones_v * lax.rsqrt(jnp.maximum(deg_o1, 1.0)))
    sB1 = _col(lax.rsqrt(jnp.maximum(deg_i1, 1.0)))
    sAB1 = _col(sA1[:, 0] * sB1[:, 0] * ones_v)

    p = params['is1']
    f1x = _tag_xla(feat, src, dst, w1, N1, params['sg1']['Ws'],
                   params['sg1']['b'])
    h1x = _prelu_xla(_tag_xla(f1x, src, dst, w1, N1, p['Ws'], p['b']), p['a'])
    summary1 = jax.nn.sigmoid(jnp.mean(h1x, axis=0))
    sc1p = (h1x @ p['Wd']) @ summary1
    scores1 = jax.nn.sigmoid(sc1p)
    sel1 = jnp.argsort(-scores1)[:K1]
    f1 = jnp.pad(f1x, ((0, NPAD - N1), (0, 0)))
    feat_origin = f1
    down1 = jnp.pad(h1x, ((0, NPAD - N1), (0, 0)))

    perm1 = jax.random.permutation(jax.random.key(1), N1)
    p1ext = jnp.concatenate(
        [perm1.astype(jnp.int32), jnp.arange(N1, NPAD, dtype=jnp.int32)])
    f1n = _perm_copy(f1, p1ext, onescol)
    hn1 = _tag_run(f1n, src, dst, sA1, sAB1, sB1, p, onescol, act=2, a=p['a'])
    v1 = p['Wd'] @ summary1
    sc1n = _matvec(hn1, v1)
    logit1 = jnp.concatenate([sc1p, sc1n[:N1]])

    maskA_b = jnp.zeros((N1,), bool).at[sel1].set(True)
    maskA = jnp.pad(maskA_b.astype(jnp.float32), (0, NPAD - N1))
    mAcol = _col(maskA)
    rank1 = jnp.zeros((N1,), jnp.int32).at[sel1].set(
        jnp.arange(K1, dtype=jnp.int32))
    ar = jnp.arange(E, dtype=jnp.int32)
    trashv = N1 + (ar & (NTRASH - 1))
    vA = maskA_b[src] & maskA_b[dst]
    srcA = jnp.where(vA, src, trashv)
    dstA = jnp.where(vA, dst, trashv)
    deg_iA = _degs(src, dstA)
    deg_oA = _degs(dst, srcA)
    sA2 = _col(maskA * lax.rsqrt(jnp.maximum(deg_oA, 1.0)))
    sB2 = _col(lax.rsqrt(jnp.maximum(deg_iA, 1.0)))
    sAB2 = _col(sA2[:, 0] * sB2[:, 0])

    f2p = _tc_scale([f1], _col(jnp.pad(scores1, (0, NPAD - N1)) * maskA))
    f2 = _tag_run(f2p, src, dstA, sA2, sAB2, sB2, params['sg2'], mAcol)

    p = params['is2']
    h2 = _tag_run(f2, src, dstA, sA2, sAB2, sB2, p, mAcol, act=2, a=p['a'])
    perm2 = jax.random.permutation(jax.random.key(2), K1)
    p2core = sel1[perm2[rank1]].astype(jnp.int32)
    p2 = jnp.where(maskA_b, p2core, jnp.arange(N1, dtype=jnp.int32))
    p2ext = jnp.concatenate([p2, jnp.arange(N1, NPAD, dtype=jnp.int32)])
    f2n = _perm_copy(f2, p2ext, onescol)
    hn2 = _tag_run(f2n, src, dstA, sA2, sAB2, sB2, p, mAcol, act=2, a=p['a'])
    v2 = _make_summary(K1)(h2, p['Wd'])
    sc2p = _matvec(h2, v2)
    sc2n = _matvec(hn2, v2)
    logit2 = jnp.concatenate([sc2p[sel1], sc2n[sel1]])
    scores2 = jax.nn.sigmoid(sc2p[sel1])
    sel2 = jnp.argsort(-scores2)[:K2]
    down2 = h2

    sel2n1 = sel1[sel2]
    maskB_b = jnp.zeros((N1,), bool).at[sel2n1].set(True)
    maskB = jnp.pad(maskB_b.astype(jnp.float32), (0, NPAD - N1))
    mBcol = _col(maskB)
    vB = maskB_b[src] & maskB_b[dst]
    srcB = jnp.where(vB, src, trashv)
    dstB = jnp.where(vB, dst, trashv)
    deg_iB = _degs(src, dstB)
    deg_oB = _degs(dst, srcB)
    sA3 = _col(maskB * lax.rsqrt(jnp.maximum(deg_oB, 1.0)))
    sB3 = _col(lax.rsqrt(jnp.maximum(deg_iB, 1.0)))
    sAB3 = _col(sA3[:, 0] * sB3[:, 0])

    sc2emb = jax.nn.sigmoid(sc2p) * maskB
    f3p = _tc_scale([f2], _col(sc2emb))

    f1a = _tag_run(f1, src, dst, sA1, sAB1, sB1, params['s1l1'], onescol, act=1)
    f2a = _tag_run(f2, src, dstA, sA2, sAB2, sB2, params['s2l1'], mAcol, act=1)
    f3a = _tag_run(f3p, src, dstB, sA3, sAB3, sB3, params['s3l1'], mBcol, act=1)

    WT = params['cf1']['W'].T
    bcf = params['cf1']['b']
    f1b = _tc_affine(f1a, WT, bcf, onescol)
    f2b = _tc_affine(f2a, WT, bcf, mAcol)
    f3b = _tc_affine(f3a, WT, bcf, mBcol)

    f1c = _tag_run(f1b, src, dst, sA1, sAB1, sB1, params['s1l2'], onescol, act=1)
    f2c = _tag_run(f2b, src, dstA, sA2, sAB2, sB2, params['s2l2'], mAcol, act=1)
    f3c = _tag_run(f3b, src, dstB, sA3, sAB3, sB3, params['s3l2'], mBcol, act=1)

    f3o = _tag_run(f3c, src, dstA, sA2, sAB2, sB2, params['end_unpool_s32'],
                   mAcol)
    S = _tc_scale([f2c, f3o, down2], onescol)
    f2out = _tag_run(S, src, dst, sA1, sAB1, sB1, params['end_unpool_s21'],
                     onescol)

    L = _tc_scale([f1c, f2out, down1], onescol)
    R = feat_origin
    WsL = [w[:HID] for w in params['end_gcn']['Ws']]
    WsR = [w[HID:] for w in params['end_gcn']['Ws']]
    uL = _tc_scale([L], sA1)
    uR = _tc_scale([R], sA1)
    y1L = _prop(uL, src, dst, HID)
    y1R = _prop(uR, src, dst, HID)
    u2L = _tc_scale([y1L], sAB1)
    u2R = _tc_scale([y1R], sAB1)
    y2L = _prop(u2L, src, dst, HID)
    y2R = _prop(u2R, src, dst, HID)
    zb = jnp.zeros((HID,), jnp.float32)
    hhalf = _tc_stage(L, y1L, y2L, sB1, onescol, jnp.stack(WsL), zb)
    fagg = _tc_stage(R, y1R, y2R, sB1, onescol, jnp.stack(WsR),
                     params['end_gcn']['b'], acc=hhalf)

    return fagg[:N1], logit1, logit2


# speedup vs baseline: 2.9382x; 1.0179x over previous
import functools

import jax
import jax.numpy as jnp
from jax import lax
from jax.experimental import pallas as pl
from jax.experimental.pallas import tpu as pltpu
from jax.experimental.pallas import tpu_sc as plsc

N1 = 10000
E = 320000
HID = 128
RATIO = 0.8
K1 = int(RATIO * N1)
K2 = int(RATIO * K1)
NTRASH = 32
NPAD = 10240
NC, NS, NW = 2, 16, 32
ROWS_PT = NPAD // NS
BR = 1024
NBLK = NPAD // BR


@functools.lru_cache(maxsize=None)
def _make_prop(D, NE, K):
    ne_w = NE // NW
    nch = ne_w // K
    assert ne_w % K == 0 and K % 8 == 0 and K <= 128
    mesh = plsc.VectorSubcoreMesh(core_axis_name="c", subcore_axis_name="s")

    def body(u_hbm, gi_hbm, si_hbm, zrow_hbm, out_hbm, gi_all, si_v, rows_v,
             acc_sh, sem_i, sem_g, sem_s):
        c = lax.axis_index("c")
        s = lax.axis_index("s")
        wid = s * NC + c
        base0 = wid * ne_w
        pltpu.sync_copy(zrow_hbm, acc_sh.at[pl.ds(s * ROWS_PT, ROWS_PT)])
        pltpu.sync_copy(gi_hbm.at[pl.ds(base0, ne_w)], gi_all)
        pltpu.sync_copy(si_hbm.at[pl.ds(base0, K)], si_v.at[0])
        plsc.subcore_barrier()
        pltpu.async_copy(u_hbm.at[gi_all.at[pl.ds(0, K)]], rows_v.at[0],
                         sem_g)
        if nch > 1:
            pltpu.async_copy(si_hbm.at[pl.ds(base0 + K, K)], si_v.at[1],
                             sem_i)
            pltpu.async_copy(u_hbm.at[gi_all.at[pl.ds(K, K)]], rows_v.at[1],
                             sem_g)

        def chunk(i, carry):
            b = lax.rem(i, 4)
            pltpu.make_async_copy(u_hbm.at[gi_all.at[pl.ds(0, K)]],
                                  rows_v.at[b], sem_g).wait()

            @pl.when(i >= 1)
            def _():
                pltpu.make_async_copy(si_hbm.at[pl.ds(base0, K)],
                                      si_v.at[b], sem_i).wait()

            pltpu.async_copy(rows_v.at[b], acc_sh.at[si_v.at[b]], sem_s,
                             add=True)

            @pl.when(i + 2 < nch)
            def _():
                nb = lax.rem(i + 2, 4)

                @pl.when(i >= 2)
                def _():
                    pltpu.make_async_copy(rows_v.at[nb],
                                          acc_sh.at[si_v.at[nb]],
                                          sem_s).wait()
                pltpu.async_copy(si_hbm.at[pl.ds(base0 + (i + 2) * K, K)],
                                 si_v.at[nb], sem_i)
                pltpu.async_copy(u_hbm.at[gi_all.at[pl.ds((i + 2) * K, K)]],
                                 rows_v.at[nb], sem_g)

            return carry

        lax.fori_loop(0, nch, chunk, 0)
        for _ in range(min(nch, 4)):
            pltpu.make_async_copy(rows_v.at[0], acc_sh.at[si_v.at[0]],
                                  sem_s).wait()
        plsc.subcore_barrier()
        pltpu.sync_copy(acc_sh.at[pl.ds(s * ROWS_PT, ROWS_PT)],
                        out_hbm.at[pl.ds(c * NPAD + s * ROWS_PT, ROWS_PT)])

    return pl.kernel(
        body,
        out_type=jax.ShapeDtypeStruct((NC * NPAD, D), jnp.float32),
        mesh=mesh,
        scratch_types=[
            pltpu.VMEM((ne_w,), jnp.int32),
            pltpu.VMEM((4, K), jnp.int32),
            pltpu.VMEM((4, K, D), jnp.float32),
            pltpu.VMEM_SHARED((NPAD, D), jnp.float32),
            pltpu.SemaphoreType.DMA,
            pltpu.SemaphoreType.DMA,
            pltpu.SemaphoreType.DMA,
        ],
        compiler_params=pltpu.CompilerParams(
            use_tc_tiling_on_sc=(D % 128 == 0)),
    )


def _prop(u, gi, si, D):
    K = 64
    ne0 = gi.shape[0]
    ne = -(-ne0 // (NW * K)) * (NW * K)
    if ne != ne0:
        ar = jnp.arange(ne - ne0, dtype=jnp.int32)
        gi = jnp.concatenate([gi, ar % N1])
        si = jnp.concatenate([si, N1 + (ar % NTRASH)])
    nch = ne // NW // K
    z = jnp.zeros((ROWS_PT, D), jnp.float32)
    out = _make_prop(D, ne, K)(u, gi, si, z)
    return out.reshape(NC, NPAD, D)


def _dot(a, b):
    return lax.dot_general(a, b, (((1,), (0,)), ((), ())),
                           preferred_element_type=jnp.float32)


_VSPEC = pl.BlockSpec((BR, 1), lambda i: (i, 0))
_XSPEC = pl.BlockSpec((BR, HID), lambda i: (i, 0))
_YSPEC = pl.BlockSpec((NC, BR, HID), lambda i: (0, i, 0))


@functools.lru_cache(maxsize=None)
def _make_stage(act, with_acc):
    def body(*refs):
        if with_acc:
            (x0, y1, y2, s, m, w, b, a, acc, o) = refs
        else:
            (x0, y1, y2, s, m, w, b, a, o) = refs
            acc = None
        sv = s[...]
        h = _dot(x0[...], w[0])
        h = h + _dot((y1[0] + y1[1]) * sv, w[1])
        h = h + _dot((y2[0] + y2[1]) * sv, w[2])
        if acc is not None:
            h = h + acc[...]
        h = (h + b[...]) * m[...]
        if act == 1:
            h = jnp.maximum(h, 0.0)
        elif act == 2:
            h = jnp.where(h >= 0.0, h, a[0, 0] * h)
        o[...] = h

    in_specs = [
        _XSPEC, _YSPEC, _YSPEC, _VSPEC, _VSPEC,
        pl.BlockSpec((3, HID, HID), lambda i: (0, 0, 0)),
        pl.BlockSpec((1, HID), lambda i: (0, 0)),
        pl.BlockSpec((1, 1), lambda i: (0, 0)),
    ]
    if with_acc:
        in_specs.append(_XSPEC)
    return pl.pallas_call(
        body,
        grid=(NBLK,),
        in_specs=in_specs,
        out_specs=_XSPEC,
        out_shape=jax.ShapeDtypeStruct((NPAD, HID), jnp.float32),
    )


def _tc_stage(x0, y1p, y2p, svec, mvec, Ws3, bias, act=0, a=0.0, acc=None):
    aarr = jnp.asarray(a, jnp.float32).reshape(1, 1)
    barr = bias.reshape(1, HID)
    args = [x0, y1p, y2p, svec, mvec, Ws3, barr, aarr]
    if acc is not None:
        args.append(acc)
    return _make_stage(act, acc is not None)(*args)


@functools.lru_cache(maxsize=None)
def _make_scale(dims_key):
    def body(*refs):
        xs, s, o = refs[:-2], refs[-2], refs[-1]
        accv = None
        for r, nd in zip(xs, dims_key):
            v = r[...]
            if nd == 3:
                v = v[0] + v[1]
            accv = v if accv is None else accv + v
        o[...] = accv * s[...]

    in_specs = [(_YSPEC if nd == 3 else _XSPEC) for nd in dims_key] + [_VSPEC]
    return pl.pallas_call(
        body,
        grid=(NBLK,),
        in_specs=in_specs,
        out_specs=_XSPEC,
        out_shape=jax.ShapeDtypeStruct((NPAD, HID), jnp.float32),
    )


def _tc_scale(xs, svec):
    dims_key = tuple(x.ndim for x in xs)
    return _make_scale(dims_key)(*xs, svec)


@functools.lru_cache(maxsize=None)
def _make_affine():
    def body(x, w, b, m, o):
        o[...] = (_dot(x[...], w[...]) + b[...]) * m[...]

    return pl.pallas_call(
        body,
        grid=(NBLK,),
        in_specs=[
            _XSPEC,
            pl.BlockSpec((HID, HID), lambda i: (0, 0)),
            pl.BlockSpec((1, HID), lambda i: (0, 0)),
            _VSPEC,
        ],
        out_specs=_XSPEC,
        out_shape=jax.ShapeDtypeStruct((NPAD, HID), jnp.float32),
    )


def _tc_affine(x, w, bias, mvec):
    return _make_affine()(x, w, bias.reshape(1, HID), mvec)


@functools.lru_cache(maxsize=None)
def _make_summary(kdiv):
    def body(h, wd, o):
        ssum = jnp.sum(h[...], axis=0, keepdims=True) * (1.0 / kdiv)
        sig = jax.nn.sigmoid(ssum)
        o[...] = lax.dot_general(sig, wd[...], (((1,), (1,)), ((), ())),
                                 preferred_element_type=jnp.float32)

    return pl.pallas_call(
        body,
        in_specs=[
            pl.BlockSpec((NPAD, HID), lambda: (0, 0)),
            pl.BlockSpec((HID, HID), lambda: (0, 0)),
        ],
        out_specs=pl.BlockSpec((1, HID), lambda: (0, 0)),
        out_shape=jax.ShapeDtypeStruct((1, HID), jnp.float32),
    )


@functools.lru_cache(maxsize=None)
def _make_matvec():
    def body(h, v, o):
        o[...] = _dot(h[...], v[...])

    return pl.pallas_call(
        body,
        grid=(NBLK,),
        in_specs=[_XSPEC, pl.BlockSpec((HID, 1), lambda i: (0, 0))],
        out_specs=_VSPEC,
        out_shape=jax.ShapeDtypeStruct((NPAD, 1), jnp.float32),
    )


def _matvec(h, v128):
    return _make_matvec()(h, v128.reshape(HID, 1))[:, 0]


def _stack_ws(Ws):
    ws = list(Ws)
    if len(ws) == 2:
        ws.append(jnp.zeros_like(ws[1]))
    return jnp.stack(ws)


def _tag_run(x0, gi, si, sA, sAB, sB, p, mvec, act=0, a=0.0):
    u0 = _tc_scale([x0], sA)
    y1 = _prop(u0, gi, si, HID)
    if len(p['Ws']) == 3:
        u1 = _tc_scale([y1], sAB)
        y2 = _prop(u1, gi, si, HID)
    else:
        y2 = jnp.zeros_like(y1)
    return _tc_stage(x0, y1, y2, sB, mvec, _stack_ws(p['Ws']), p['b'], act, a)


def _degs(gi, si):
    ones16 = jnp.ones((NPAD, 16), jnp.float32)
    out = _prop(ones16, gi, si, 16)
    return out[0, :, 0] + out[1, :, 0]


def _perm_copy(x, pext, onescol):
    yp = _prop(x, pext, jnp.arange(NPAD, dtype=jnp.int32), HID)
    return _tc_scale([yp], onescol)


def _col(v):
    return v.reshape(NPAD, 1)


def _tag_xla(x, src, dst, w, n, Ws, b):
    deg_i = jnp.maximum(jnp.zeros(n, x.dtype).at[dst].add(w), 1.0)
    deg_o = jnp.maximum(jnp.zeros(n, x.dtype).at[src].add(w), 1.0)
    norm = w / jnp.sqrt(jnp.take(deg_o, src) * jnp.take(deg_i, dst))
    h = x @ Ws[0]
    xk = x
    for Wk in Ws[1:]:
        xk = jnp.zeros((n, xk.shape[1]), x.dtype).at[dst].add(
            norm[:, None] * jnp.take(xk, src, axis=0))
        h = h + xk @ Wk
    return h + b


def _prelu_xla(x, a):
    return jnp.where(x >= 0, x, a * x)


def kernel(feat, edge_index, params):
    src = edge_index[0]
    dst = edge_index[1]
    featP = jnp.pad(feat, ((0, NPAD - N1), (0, 0)))
    ones_v = (jnp.arange(NPAD) < N1).astype(jnp.float32)
    onescol = _col(ones_v)

    w1 = jnp.ones((E,), jnp.float32)
    deg_i1 = jnp.pad(jnp.zeros((N1,), jnp.float32).at[dst].add(w1),
                     (0, NPAD - N1))
    deg_o1 = jnp.pad(jnp.zeros((N1,), jnp.float32).at[src].add(w1),
                     (0, NPAD - N1))
    sA1 = _col(ones_v * lax.rsqrt(jnp.maximum(deg_o1, 1.0)))
    sB1 = _col(lax.rsqrt(jnp.maximum(deg_i1, 1.0)))
    sAB1 = _col(sA1[:, 0] * sB1[:, 0] * ones_v)

    p = params['is1']
    f1x = _tag_xla(feat, src, dst, w1, N1, params['sg1']['Ws'],
                   params['sg1']['b'])
    h1x = _prelu_xla(_tag_xla(f1x, src, dst, w1, N1, p['Ws'], p['b']), p['a'])
    summary1 = jax.nn.sigmoid(jnp.mean(h1x, axis=0))
    sc1p = (h1x @ p['Wd']) @ summary1
    scores1 = jax.nn.sigmoid(sc1p)
    sel1 = jnp.argsort(-scores1)[:K1]
    f1 = jnp.pad(f1x, ((0, NPAD - N1), (0, 0)))
    feat_origin = f1
    down1 = jnp.pad(h1x, ((0, NPAD - N1), (0, 0)))

    perm1 = jax.random.permutation(jax.random.key(1), N1)
    p1ext = jnp.concatenate(
        [perm1.astype(jnp.int32), jnp.arange(N1, NPAD, dtype=jnp.int32)])
    f1n = _perm_copy(f1, p1ext, onescol)
    hn1 = _tag_run(f1n, src, dst, sA1, sAB1, sB1, p, onescol, act=2, a=p['a'])
    v1 = p['Wd'] @ summary1
    sc1n = _matvec(hn1, v1)
    logit1 = jnp.concatenate([sc1p, sc1n[:N1]])

    maskA_b = jnp.zeros((N1,), bool).at[sel1].set(True)
    maskA = jnp.pad(maskA_b.astype(jnp.float32), (0, NPAD - N1))
    mAcol = _col(maskA)
    rank1 = jnp.zeros((N1,), jnp.int32).at[sel1].set(
        jnp.arange(K1, dtype=jnp.int32))
    ar = jnp.arange(E, dtype=jnp.int32)
    trashv = N1 + (ar & (NTRASH - 1))
    vA = maskA_b[src] & maskA_b[dst]
    srcA = jnp.where(vA, src, trashv)
    dstA = jnp.where(vA, dst, trashv)
    deg_iA = _degs(src, dstA)
    deg_oA = _degs(dst, srcA)
    sA2 = _col(maskA * lax.rsqrt(jnp.maximum(deg_oA, 1.0)))
    sB2 = _col(lax.rsqrt(jnp.maximum(deg_iA, 1.0)))
    sAB2 = _col(sA2[:, 0] * sB2[:, 0])

    f2p = _tc_scale([f1], _col(jnp.pad(scores1, (0, NPAD - N1)) * maskA))
    f2 = _tag_run(f2p, src, dstA, sA2, sAB2, sB2, params['sg2'], mAcol)

    p = params['is2']
    h2 = _tag_run(f2, src, dstA, sA2, sAB2, sB2, p, mAcol, act=2, a=p['a'])
    perm2 = jax.random.permutation(jax.random.key(2), K1)
    p2core = sel1[perm2[rank1]].astype(jnp.int32)
    p2 = jnp.where(maskA_b, p2core, jnp.arange(N1, dtype=jnp.int32))
    p2ext = jnp.concatenate([p2, jnp.arange(N1, NPAD, dtype=jnp.int32)])
    f2n = _perm_copy(f2, p2ext, onescol)
    hn2 = _tag_run(f2n, src, dstA, sA2, sAB2, sB2, p, mAcol, act=2, a=p['a'])
    v2 = _make_summary(K1)(h2, p['Wd'])
    sc2p = _matvec(h2, v2)
    sc2n = _matvec(hn2, v2)
    logit2 = jnp.concatenate([sc2p[sel1], sc2n[sel1]])
    scores2 = jax.nn.sigmoid(sc2p[sel1])
    sel2 = jnp.argsort(-scores2)[:K2]
    down2 = h2

    sel2n1 = sel1[sel2]
    maskB_b = jnp.zeros((N1,), bool).at[sel2n1].set(True)
    maskB = jnp.pad(maskB_b.astype(jnp.float32), (0, NPAD - N1))
    mBcol = _col(maskB)
    vB = maskB_b[src] & maskB_b[dst]
    srcB = jnp.where(vB, src, trashv)
    dstB = jnp.where(vB, dst, trashv)
    deg_iB = _degs(src, dstB)
    deg_oB = _degs(dst, srcB)
    sA3 = _col(maskB * lax.rsqrt(jnp.maximum(deg_oB, 1.0)))
    sB3 = _col(lax.rsqrt(jnp.maximum(deg_iB, 1.0)))
    sAB3 = _col(sA3[:, 0] * sB3[:, 0])

    sc2emb = jax.nn.sigmoid(sc2p) * maskB
    f3p = _tc_scale([f2], _col(sc2emb))

    f1a = _tag_run(f1, src, dst, sA1, sAB1, sB1, params['s1l1'], onescol, act=1)
    f2a = _tag_run(f2, src, dstA, sA2, sAB2, sB2, params['s2l1'], mAcol, act=1)
    f3a = _tag_run(f3p, src, dstB, sA3, sAB3, sB3, params['s3l1'], mBcol, act=1)

    WT = params['cf1']['W'].T
    bcf = params['cf1']['b']
    f1b = _tc_affine(f1a, WT, bcf, onescol)
    f2b = _tc_affine(f2a, WT, bcf, mAcol)
    f3b = _tc_affine(f3a, WT, bcf, mBcol)

    f1c = _tag_run(f1b, src, dst, sA1, sAB1, sB1, params['s1l2'], onescol, act=1)
    f2c = _tag_run(f2b, src, dstA, sA2, sAB2, sB2, params['s2l2'], mAcol, act=1)
    f3c = _tag_run(f3b, src, dstB, sA3, sAB3, sB3, params['s3l2'], mBcol, act=1)

    f3o = _tag_run(f3c, src, dstA, sA2, sAB2, sB2, params['end_unpool_s32'],
                   mAcol)
    S = _tc_scale([f2c, f3o, down2], onescol)
    f2out = _tag_run(S, src, dst, sA1, sAB1, sB1, params['end_unpool_s21'],
                     onescol)

    L = _tc_scale([f1c, f2out, down1], onescol)
    R = feat_origin
    WsL = [w[:HID] for w in params['end_gcn']['Ws']]
    WsR = [w[HID:] for w in params['end_gcn']['Ws']]
    uL = _tc_scale([L], sA1)
    uR = _tc_scale([R], sA1)
    y1L = _prop(uL, src, dst, HID)
    y1R = _prop(uR, src, dst, HID)
    u2L = _tc_scale([y1L], sAB1)
    u2R = _tc_scale([y1R], sAB1)
    y2L = _prop(u2L, src, dst, HID)
    y2R = _prop(u2R, src, dst, HID)
    zb = jnp.zeros((HID,), jnp.float32)
    hhalf = _tc_stage(L, y1L, y2L, sB1, onescol, jnp.stack(WsL), zb)
    fagg = _tc_stage(R, y1R, y2R, sB1, onescol, jnp.stack(WsR),
                     params['end_gcn']['b'], acc=hhalf)

    return fagg[:N1], logit1, logit2
